# Initial kernel scaffold; baseline (speedup 1.0000x reference)
import jax
import jax.numpy as jnp
from jax.experimental import pallas as pl


def kernel(hidden_states, router_logits, w13_weight, w2_weight, w13_input_scale, w2_input_scale, w13_weight_scale, w2_weight_scale):
    raise NotImplementedError("write your pallas kernel here")


# speedup vs baseline: 1.5709x; 1.5709x over previous
import functools

import jax
import jax.numpy as jnp
from jax.experimental import pallas as pl
from jax.experimental.pallas import tpu as pltpu

NUM_EXPERTS = 8
TOP_K = 2
HIDDEN = 1024
INTER = 1024
TOKENS = 2048

T_TILE = 512


def _combine_col(logits, e):
    l = logits.astype(jnp.float32)
    tt = l.shape[0]
    col_ids = jax.lax.broadcasted_iota(jnp.int32, l.shape, 1)
    v1 = jnp.max(l, axis=1, keepdims=True)
    is_max1 = l == v1
    first1 = jnp.min(jnp.where(is_max1, col_ids, NUM_EXPERTS), axis=1, keepdims=True)
    top1 = col_ids == first1
    l2 = jnp.where(top1, -jnp.inf, l)
    v2 = jnp.max(l2, axis=1, keepdims=True)
    is_max2 = l2 == v2
    first2 = jnp.min(jnp.where(is_max2, col_ids, NUM_EXPERTS), axis=1, keepdims=True)
    top2 = col_ids == first2
    w1 = 1.0 / (1.0 + jnp.exp(v2 - v1))
    w2 = 1.0 - w1
    combine = jnp.where(top1, w1, jnp.where(top2, w2, 0.0))
    return jnp.sum(jnp.where(col_ids == e, combine, 0.0), axis=1, keepdims=True)


def _moe_body(s1_ref, s2_ref, x_ref, logits_ref, w13_ref, w2_ref, out_ref):
    e = pl.program_id(1)
    x = x_ref[...]
    w13 = w13_ref[0]
    w2 = w2_ref[0]
    gateup = jax.lax.dot_general(
        x, w13, (((1,), (1,)), ((), ())),
        preferred_element_type=jnp.float32)
    gateup = gateup * s1_ref[e]
    gate = gateup[:, :INTER]
    up = gateup[:, INTER:]
    act = gate * (1.0 / (1.0 + jnp.exp(-gate))) * up
    down = jax.lax.dot_general(
        act, w2, (((1,), (1,)), ((), ())),
        preferred_element_type=jnp.float32)
    down = down * s2_ref[e]
    contrib = down * _combine_col(logits_ref[...], e)

    @pl.when(e == 0)
    def _init():
        out_ref[...] = contrib

    @pl.when(e != 0)
    def _acc():
        out_ref[...] = out_ref[...] + contrib


@functools.partial(jax.jit, static_argnames=())
def kernel(hidden_states, router_logits, w13_weight, w2_weight,
           w13_input_scale, w2_input_scale, w13_weight_scale, w2_weight_scale):
    s1 = (w13_input_scale * w13_weight_scale).astype(jnp.float32)
    s2 = (w2_input_scale * w2_weight_scale).astype(jnp.float32)
    n_t = TOKENS // T_TILE
    grid = (n_t, NUM_EXPERTS)
    out = pl.pallas_call(
        _moe_body,
        grid_spec=pltpu.PrefetchScalarGridSpec(
            num_scalar_prefetch=2,
            grid=grid,
            in_specs=[
                pl.BlockSpec((T_TILE, HIDDEN), lambda t, e, s1, s2: (t, 0)),
                pl.BlockSpec((T_TILE, NUM_EXPERTS), lambda t, e, s1, s2: (t, 0)),
                pl.BlockSpec((1, 2 * INTER, HIDDEN), lambda t, e, s1, s2: (e, 0, 0)),
                pl.BlockSpec((1, HIDDEN, INTER), lambda t, e, s1, s2: (e, 0, 0)),
            ],
            out_specs=pl.BlockSpec((T_TILE, HIDDEN), lambda t, e, s1, s2: (t, 0)),
        ),
        out_shape=jax.ShapeDtypeStruct((TOKENS, HIDDEN), jnp.float32),
        compiler_params=pltpu.CompilerParams(
            dimension_semantics=("parallel", "arbitrary"),
        ),
    )(s1, s2, hidden_states.astype(jnp.float32), router_logits, w13_weight, w2_weight)
    return out
